# baseline (speedup 1.0000x reference)
import functools

import jax
import jax.numpy as jnp
from jax import lax
from jax.experimental import pallas as pl
from jax.experimental.pallas import tpu as pltpu
from jax.experimental.pallas import tpu_sc as plsc

B, N, C, K = 8, 2048, 256, 16
NW = 32
ROWS_PER_W = (B * N) // NW
CH = 8
NCHUNK = ROWS_PER_W // CH


def _rne_hi16(f):
    u = lax.bitcast_convert_type(f, jnp.int32)
    return u + jnp.int32(0x7FFF) + ((u >> 16) & 1)


AR = 1024


def _mm_body(x_ref, wlo_ref, whi_ref, o_ref):
    k = pl.program_id(1)
    x = x_ref[...]
    co = pl.multiple_of(k * (C // 2), C // 2)
    lo = jnp.dot(x, wlo_ref[:, pl.ds(co, C // 2)],
                 preferred_element_type=jnp.float32)
    hi = jnp.dot(x, whi_ref[:, pl.ds(co, C // 2)],
                 preferred_element_type=jnp.float32)
    word = (_rne_hi16(hi) & jnp.int32(-65536)) | \
           ((_rne_hi16(lo) >> 16) & jnp.int32(0xFFFF))
    o_ref[...] = word


def _stage_a(x2d, wlo, whi):
    nblk = B * N // AR
    return pl.pallas_call(
        _mm_body,
        grid=(nblk, K),
        in_specs=[
            pl.BlockSpec((AR, C), lambda i, k: (i, 0)),
            pl.BlockSpec((C, K * C // 2), lambda i, k: (0, 0)),
            pl.BlockSpec((C, K * C // 2), lambda i, k: (0, 0)),
        ],
        out_specs=pl.BlockSpec((AR, C // 2), lambda i, k: (k * nblk + i, 0)),
        out_shape=jax.ShapeDtypeStruct((B * N * K, C // 2), jnp.int32),
    )(x2d, wlo, whi)


def _half_channels(off):
    s = []
    for c in range(C // 32):
        s.extend(range(32 * c + off, 32 * c + off + 16))
    return jnp.array(s, dtype=jnp.int32)


def _sc_body(yg_hbm, gidx_hbm, out_hbm,
             idx_all, rows0, rows1, acc, sem0, sem1):
    wid = lax.axis_index("s") * 2 + lax.axis_index("c")
    row_base = wid * ROWS_PER_W

    pltpu.sync_copy(
        gidx_hbm.at[pl.ds(pl.multiple_of(row_base * K, 128), ROWS_PER_W * K)],
        idx_all)

    def issue(ch, rows_v, sem):
        idx_v = idx_all.at[pl.ds(pl.multiple_of(ch * CH * K, 128), CH * K)]
        pltpu.make_async_copy(yg_hbm.at[idx_v], rows_v, sem).start()

    def wait(ch, rows_v, sem):
        idx_v = idx_all.at[pl.ds(pl.multiple_of(ch * CH * K, 128), CH * K)]
        pltpu.make_async_copy(yg_hbm.at[idx_v], rows_v, sem).wait()

    def unpack2(vi):
        e = lax.bitcast_convert_type(vi << 16, jnp.float32)
        o = lax.bitcast_convert_type(vi & jnp.int32(-65536), jnp.float32)
        return e, o

    def reduce_store(ch, rows_v):
        def rbody(r, _):
            for c in range(C // 32):
                co = c * 16
                e, o = unpack2(rows_v[r * K, pl.ds(co, 16)])
                for k in range(1, K):
                    ek, ok = unpack2(rows_v[r * K + k, pl.ds(co, 16)])
                    e = e + ek
                    o = o + ok
                acc[r, pl.ds(2 * co, 16)] = e
                acc[r, pl.ds(2 * co + 16, 16)] = o
            return 0
        lax.fori_loop(0, CH, rbody, 0)
        dst = pl.multiple_of(row_base + ch * CH, 8)
        pltpu.sync_copy(acc, out_hbm.at[pl.ds(dst, CH)])

    issue(0, rows0, sem0)

    def gbody(g, _):
        ch0 = 2 * g
        issue(ch0 + 1, rows1, sem1)
        wait(ch0, rows0, sem0)
        reduce_store(ch0, rows0)

        @pl.when(g + 1 < NCHUNK // 2)
        def _():
            issue(ch0 + 2, rows0, sem0)

        wait(ch0 + 1, rows1, sem1)
        reduce_store(ch0 + 1, rows1)
        return 0

    lax.fori_loop(0, NCHUNK // 2, gbody, 0)


def _stage_b(yg_flat, gidx_flat):
    mesh = plsc.VectorSubcoreMesh(core_axis_name="c", subcore_axis_name="s")
    return pl.kernel(
        _sc_body,
        out_type=jax.ShapeDtypeStruct((B * N, C), jnp.float32),
        mesh=mesh,
        scratch_types=[
            pltpu.VMEM((ROWS_PER_W * K,), jnp.int32),
            pltpu.VMEM((CH * K, C // 2), jnp.int32),
            pltpu.VMEM((CH * K, C // 2), jnp.int32),
            pltpu.VMEM((CH, C), jnp.float32),
            pltpu.SemaphoreType.DMA,
            pltpu.SemaphoreType.DMA,
        ],
    )(yg_flat, gidx_flat)


def _c1_body(x_ref, cp_ref, wc0_ref, w2t_ref, bc_ref, b2_ref, cnn_ref):
    x = x_ref[...]
    conv = (cp_ref[...] + bc_ref[...]
            + jnp.dot(x, wc0_ref[...], preferred_element_type=jnp.float32))
    cnn_ref[...] = jnp.dot(conv, w2t_ref[...],
                           preferred_element_type=jnp.float32) + b2_ref[...]


def _stage_c1(x2d, cp, wc0t, w2t, bc2, b22):
    return pl.pallas_call(
        _c1_body,
        grid=(B * N // 256,),
        in_specs=[
            pl.BlockSpec((256, C), lambda i: (i, 0)),
            pl.BlockSpec((256, C), lambda i: (i, 0)),
            pl.BlockSpec((C, C), lambda i: (0, 0)),
            pl.BlockSpec((C, C), lambda i: (0, 0)),
            pl.BlockSpec((1, C), lambda i: (0, 0)),
            pl.BlockSpec((1, C), lambda i: (0, 0)),
        ],
        out_specs=pl.BlockSpec((256, C), lambda i: (i, 0)),
        out_shape=jax.ShapeDtypeStruct((B * N, C), jnp.float32),
    )(x2d, cp, wc0t, w2t, bc2, b22)


def _c2_body(col_ref, val_ref, imp_ref):
    i = pl.program_id(1)
    row = lax.broadcasted_iota(jnp.int32, (256, 128), 0) + i * 256
    cols = col_ref[0]
    vals = val_ref[0]
    for s in range(N // 128):
        col = lax.broadcasted_iota(jnp.int32, (256, 128), 1) + s * 128
        imp = jnp.where(col == row, 1.0, 0.0).astype(jnp.float32)
        for k in range(K):
            imp = jnp.where(col == cols[:, k:k + 1], vals[:, k:k + 1], imp)
        imp_ref[0, :, pl.ds(s * 128, 128)] = imp


def _stage_c2(scol3, sval3):
    return pl.pallas_call(
        _c2_body,
        grid=(B, N // 256),
        in_specs=[
            pl.BlockSpec((1, 256, K), lambda b, i: (b, i, 0)),
            pl.BlockSpec((1, 256, K), lambda b, i: (b, i, 0)),
        ],
        out_specs=pl.BlockSpec((1, 256, N), lambda b, i: (b, i, 0)),
        out_shape=jax.ShapeDtypeStruct((B, N, N), jnp.float32),
    )(scol3, sval3)


def kernel(x, adj_mat, W_conv, b_conv, W2, b2):
    adj = adj_mat.astype(jnp.int32)
    wg3 = jnp.transpose(W_conv[:, :, 1:], (1, 2, 0))
    wlo = wg3[:, :, _half_channels(0)].reshape(C, K * C // 2)
    whi = wg3[:, :, _half_channels(16)].reshape(C, K * C // 2)
    wc0t = W_conv[:, :, 0].T
    w2t = W2.T
    bc2 = b_conv.reshape(1, C)
    b22 = b2.reshape(1, C)

    x2d = x.reshape(B * N, C)
    yg_flat = _stage_a(x2d, wlo, whi)

    gidx = (adj + (jnp.arange(B, dtype=jnp.int32) * N)[:, None, None]
            + (jnp.arange(K, dtype=jnp.int32) * (B * N))[None, None, :])
    cp = _stage_b(yg_flat, gidx.reshape(-1))

    cnn2d = _stage_c1(x2d, cp, wc0t, w2t, bc2, b22)

    rowid = jnp.arange(B * N, dtype=jnp.int32)[:, None]
    keys = (rowid * N + adj.reshape(B * N, K)).reshape(-1)
    perm = jnp.arange(B * N * K, dtype=jnp.int32)
    sk, sp = lax.sort_key_val(keys, perm, is_stable=False)
    nw = cnn2d[:, :K].reshape(-1)
    sv = nw[sp]
    scol = sk.reshape(B * N, K) - rowid * N
    imp = _stage_c2(scol.reshape(B, N, K), sv.reshape(B, N, K))

    return (cnn2d.reshape(B, N, C), imp)


# speedup vs baseline: 1.0095x; 1.0095x over previous
import functools

import jax
import jax.numpy as jnp
from jax import lax
from jax.experimental import pallas as pl
from jax.experimental.pallas import tpu as pltpu
from jax.experimental.pallas import tpu_sc as plsc

B, N, C, K = 8, 2048, 256, 16
NW = 32
ROWS_PER_W = (B * N) // NW
CH = 8
NCHUNK = ROWS_PER_W // CH


def _rne_hi16(f):
    u = lax.bitcast_convert_type(f, jnp.int32)
    return u + jnp.int32(0x7FFF) + ((u >> 16) & 1)


AR = 1024


def _mm_body(x_ref, wcat_ref, o_ref):
    k = pl.program_id(1)
    x = x_ref[...]
    co = pl.multiple_of(k * C, C)
    y = jnp.dot(x, wcat_ref[:, pl.ds(co, C)],
                preferred_element_type=jnp.float32)
    lo = y[:, :C // 2]
    hi = y[:, C // 2:]
    word = (_rne_hi16(hi) & jnp.int32(-65536)) | \
           ((_rne_hi16(lo) >> 16) & jnp.int32(0xFFFF))
    o_ref[...] = word


def _stage_a(x2d, wcat):
    nblk = B * N // AR
    return pl.pallas_call(
        _mm_body,
        grid=(nblk, K),
        in_specs=[
            pl.BlockSpec((AR, C), lambda i, k: (i, 0)),
            pl.BlockSpec((C, K * C), lambda i, k: (0, 0)),
        ],
        out_specs=pl.BlockSpec((AR, C // 2), lambda i, k: (k * nblk + i, 0)),
        out_shape=jax.ShapeDtypeStruct((B * N * K, C // 2), jnp.int32),
    )(x2d, wcat)


def _half_channels(off):
    s = []
    for c in range(C // 32):
        s.extend(range(32 * c + off, 32 * c + off + 16))
    return jnp.array(s, dtype=jnp.int32)


def _sc_body(yg_hbm, gidx_hbm, out_hbm,
             idx_all, rows0, rows1, acc, sem0, sem1):
    wid = lax.axis_index("s") * 2 + lax.axis_index("c")
    row_base = wid * ROWS_PER_W

    pltpu.sync_copy(
        gidx_hbm.at[pl.ds(pl.multiple_of(row_base * K, 128), ROWS_PER_W * K)],
        idx_all)

    def issue(ch, rows_v, sem):
        idx_v = idx_all.at[pl.ds(pl.multiple_of(ch * CH * K, 128), CH * K)]
        pltpu.make_async_copy(yg_hbm.at[idx_v], rows_v, sem).start()

    def wait(ch, rows_v, sem):
        idx_v = idx_all.at[pl.ds(pl.multiple_of(ch * CH * K, 128), CH * K)]
        pltpu.make_async_copy(yg_hbm.at[idx_v], rows_v, sem).wait()

    def unpack2(vi):
        e = lax.bitcast_convert_type(vi << 16, jnp.float32)
        o = lax.bitcast_convert_type(vi & jnp.int32(-65536), jnp.float32)
        return e, o

    def reduce_store(ch, rows_v):
        def rbody(r, _):
            for c in range(C // 32):
                co = c * 16
                e, o = unpack2(rows_v[r * K, pl.ds(co, 16)])
                for k in range(1, K):
                    ek, ok = unpack2(rows_v[r * K + k, pl.ds(co, 16)])
                    e = e + ek
                    o = o + ok
                acc[r, pl.ds(2 * co, 16)] = e
                acc[r, pl.ds(2 * co + 16, 16)] = o
            return 0
        lax.fori_loop(0, CH, rbody, 0)
        dst = pl.multiple_of(row_base + ch * CH, 8)
        pltpu.sync_copy(acc, out_hbm.at[pl.ds(dst, CH)])

    issue(0, rows0, sem0)

    def gbody(g, _):
        ch0 = 2 * g
        issue(ch0 + 1, rows1, sem1)
        wait(ch0, rows0, sem0)
        reduce_store(ch0, rows0)

        @pl.when(g + 1 < NCHUNK // 2)
        def _():
            issue(ch0 + 2, rows0, sem0)

        wait(ch0 + 1, rows1, sem1)
        reduce_store(ch0 + 1, rows1)
        return 0

    lax.fori_loop(0, NCHUNK // 2, gbody, 0)


def _stage_b(yg_flat, gidx_flat):
    mesh = plsc.VectorSubcoreMesh(core_axis_name="c", subcore_axis_name="s")
    return pl.kernel(
        _sc_body,
        out_type=jax.ShapeDtypeStruct((B * N, C), jnp.float32),
        mesh=mesh,
        scratch_types=[
            pltpu.VMEM((ROWS_PER_W * K,), jnp.int32),
            pltpu.VMEM((CH * K, C // 2), jnp.int32),
            pltpu.VMEM((CH * K, C // 2), jnp.int32),
            pltpu.VMEM((CH, C), jnp.float32),
            pltpu.SemaphoreType.DMA,
            pltpu.SemaphoreType.DMA,
        ],
    )(yg_flat, gidx_flat)


def _c1_body(x_ref, cp_ref, wc0_ref, w2t_ref, bc_ref, b2_ref, cnn_ref):
    x = x_ref[...]
    conv = (cp_ref[...] + bc_ref[...]
            + jnp.dot(x, wc0_ref[...], preferred_element_type=jnp.float32))
    cnn_ref[...] = jnp.dot(conv, w2t_ref[...],
                           preferred_element_type=jnp.float32) + b2_ref[...]


def _stage_c1(x2d, cp, wc0t, w2t, bc2, b22):
    return pl.pallas_call(
        _c1_body,
        grid=(B * N // 256,),
        in_specs=[
            pl.BlockSpec((256, C), lambda i: (i, 0)),
            pl.BlockSpec((256, C), lambda i: (i, 0)),
            pl.BlockSpec((C, C), lambda i: (0, 0)),
            pl.BlockSpec((C, C), lambda i: (0, 0)),
            pl.BlockSpec((1, C), lambda i: (0, 0)),
            pl.BlockSpec((1, C), lambda i: (0, 0)),
        ],
        out_specs=pl.BlockSpec((256, C), lambda i: (i, 0)),
        out_shape=jax.ShapeDtypeStruct((B * N, C), jnp.float32),
    )(x2d, cp, wc0t, w2t, bc2, b22)


def _c2_body(col_ref, val_ref, imp_ref):
    i = pl.program_id(1)
    row = lax.broadcasted_iota(jnp.int32, (256, 128), 0) + i * 256
    cols = col_ref[0]
    vals = val_ref[0]
    for s in range(N // 128):
        col = lax.broadcasted_iota(jnp.int32, (256, 128), 1) + s * 128
        imp = jnp.where(col == row, 1.0, 0.0).astype(jnp.float32)
        for k in range(K):
            imp = jnp.where(col == cols[:, k:k + 1], vals[:, k:k + 1], imp)
        imp_ref[0, :, pl.ds(s * 128, 128)] = imp


def _stage_c2(scol3, sval3):
    return pl.pallas_call(
        _c2_body,
        grid=(B, N // 256),
        in_specs=[
            pl.BlockSpec((1, 256, K), lambda b, i: (b, i, 0)),
            pl.BlockSpec((1, 256, K), lambda b, i: (b, i, 0)),
        ],
        out_specs=pl.BlockSpec((1, 256, N), lambda b, i: (b, i, 0)),
        out_shape=jax.ShapeDtypeStruct((B, N, N), jnp.float32),
    )(scol3, sval3)


def kernel(x, adj_mat, W_conv, b_conv, W2, b2):
    adj = adj_mat.astype(jnp.int32)
    wg3 = jnp.transpose(W_conv[:, :, 1:], (1, 2, 0))
    wcat = jnp.concatenate([wg3[:, :, _half_channels(0)],
                            wg3[:, :, _half_channels(16)]],
                           axis=-1).reshape(C, K * C)
    wc0t = W_conv[:, :, 0].T
    w2t = W2.T
    bc2 = b_conv.reshape(1, C)
    b22 = b2.reshape(1, C)

    x2d = x.reshape(B * N, C)
    yg_flat = _stage_a(x2d, wcat)

    gidx = (adj + (jnp.arange(B, dtype=jnp.int32) * N)[:, None, None]
            + (jnp.arange(K, dtype=jnp.int32) * (B * N))[None, None, :])
    cp = _stage_b(yg_flat, gidx.reshape(-1))

    cnn2d = _stage_c1(x2d, cp, wc0t, w2t, bc2, b22)

    rowid = jnp.arange(B * N, dtype=jnp.int32)[:, None]
    keys = (rowid * N + adj.reshape(B * N, K)).reshape(-1)
    perm = jnp.arange(B * N * K, dtype=jnp.int32)
    sk, sp = lax.sort_key_val(keys, perm, is_stable=False)
    nw = cnn2d[:, :K].reshape(-1)
    sv = nw[sp]
    scol = sk.reshape(B * N, K) - rowid * N
    imp = _stage_c2(scol.reshape(B, N, K), sv.reshape(B, N, K))

    return (cnn2d.reshape(B, N, C), imp)


# speedup vs baseline: 1.0096x; 1.0001x over previous
import jax
import jax.numpy as jnp
from jax import lax
from jax.experimental import pallas as pl
from jax.experimental.pallas import tpu as pltpu
from jax.experimental.pallas import tpu_sc as plsc

B, N, C, K = 8, 2048, 256, 16
NW = 32
ROWS_PER_W = (B * N) // NW
CH = 8
NCHUNK = ROWS_PER_W // CH


def _rne_hi16(f):
    u = lax.bitcast_convert_type(f, jnp.int32)
    return u + jnp.int32(0x7FFF) + ((u >> 16) & 1)


AR = 1024


def _mm_body(x_ref, wcat_ref, o_ref):
    k = pl.program_id(1)
    x = x_ref[...]
    co = pl.multiple_of(k * C, C)
    y = jnp.dot(x, wcat_ref[:, pl.ds(co, C)],
                preferred_element_type=jnp.float32)
    lo = y[:, :C // 2]
    hi = y[:, C // 2:]
    word = (_rne_hi16(hi) & jnp.int32(-65536)) | \
           ((_rne_hi16(lo) >> 16) & jnp.int32(0xFFFF))
    o_ref[...] = word


def _stage_a(x2d, wcat):
    nblk = B * N // AR
    return pl.pallas_call(
        _mm_body,
        grid=(nblk, K),
        in_specs=[
            pl.BlockSpec((AR, C), lambda i, k: (i, 0)),
            pl.BlockSpec((C, K * C), lambda i, k: (0, 0)),
        ],
        out_specs=pl.BlockSpec((AR, C // 2), lambda i, k: (k * nblk + i, 0)),
        out_shape=jax.ShapeDtypeStruct((B * N * K, C // 2), jnp.int32),
    )(x2d, wcat)


def _half_channels(off):
    s = []
    for c in range(C // 32):
        s.extend(range(32 * c + off, 32 * c + off + 16))
    return jnp.array(s, dtype=jnp.int32)


def _sc_body(yg_hbm, gidx_hbm, out_hbm,
             idx_all, rows0, rows1, acc, sem0, sem1):
    wid = lax.axis_index("s") * 2 + lax.axis_index("c")
    row_base = wid * ROWS_PER_W

    pltpu.sync_copy(
        gidx_hbm.at[pl.ds(pl.multiple_of(row_base * K, 128), ROWS_PER_W * K)],
        idx_all)

    def issue(ch, rows_v, sem):
        idx_v = idx_all.at[pl.ds(pl.multiple_of(ch * CH * K, 128), CH * K)]
        pltpu.make_async_copy(yg_hbm.at[idx_v], rows_v, sem).start()

    def wait(ch, rows_v, sem):
        idx_v = idx_all.at[pl.ds(pl.multiple_of(ch * CH * K, 128), CH * K)]
        pltpu.make_async_copy(yg_hbm.at[idx_v], rows_v, sem).wait()

    def unpack2(vi):
        e = lax.bitcast_convert_type(vi << 16, jnp.float32)
        o = lax.bitcast_convert_type(vi & jnp.int32(-65536), jnp.float32)
        return e, o

    def reduce_store(ch, rows_v):
        def rbody(r, _):
            for c in range(C // 32):
                co = c * 16
                e, o = unpack2(rows_v[r * K, pl.ds(co, 16)])
                for k in range(1, K):
                    ek, ok = unpack2(rows_v[r * K + k, pl.ds(co, 16)])
                    e = e + ek
                    o = o + ok
                acc[r, pl.ds(2 * co, 16)] = e
                acc[r, pl.ds(2 * co + 16, 16)] = o
            return 0
        lax.fori_loop(0, CH, rbody, 0)
        dst = pl.multiple_of(row_base + ch * CH, 8)
        pltpu.sync_copy(acc, out_hbm.at[pl.ds(dst, CH)])

    issue(0, rows0, sem0)

    def gbody(g, _):
        ch0 = 2 * g
        issue(ch0 + 1, rows1, sem1)
        wait(ch0, rows0, sem0)
        reduce_store(ch0, rows0)

        @pl.when(g + 1 < NCHUNK // 2)
        def _():
            issue(ch0 + 2, rows0, sem0)

        wait(ch0 + 1, rows1, sem1)
        reduce_store(ch0 + 1, rows1)
        return 0

    lax.fori_loop(0, NCHUNK // 2, gbody, 0)


def _stage_b(yg_flat, gidx_flat):
    mesh = plsc.VectorSubcoreMesh(core_axis_name="c", subcore_axis_name="s")
    return pl.kernel(
        _sc_body,
        out_type=jax.ShapeDtypeStruct((B * N, C), jnp.float32),
        mesh=mesh,
        scratch_types=[
            pltpu.VMEM((ROWS_PER_W * K,), jnp.int32),
            pltpu.VMEM((CH * K, C // 2), jnp.int32),
            pltpu.VMEM((CH * K, C // 2), jnp.int32),
            pltpu.VMEM((CH, C), jnp.float32),
            pltpu.SemaphoreType.DMA,
            pltpu.SemaphoreType.DMA,
        ],
    )(yg_flat, gidx_flat)


def _c1_body(x_ref, cp_ref, wc0_ref, w2t_ref, bc_ref, b2_ref, cnn_ref):
    x = x_ref[...]
    conv = (cp_ref[...] + bc_ref[...]
            + jnp.dot(x, wc0_ref[...], preferred_element_type=jnp.float32))
    cnn_ref[...] = jnp.dot(conv, w2t_ref[...],
                           preferred_element_type=jnp.float32) + b2_ref[...]


def _stage_c1(x2d, cp, wc0t, w2t, bc2, b22):
    return pl.pallas_call(
        _c1_body,
        grid=(B * N // 256,),
        in_specs=[
            pl.BlockSpec((256, C), lambda i: (i, 0)),
            pl.BlockSpec((256, C), lambda i: (i, 0)),
            pl.BlockSpec((C, C), lambda i: (0, 0)),
            pl.BlockSpec((C, C), lambda i: (0, 0)),
            pl.BlockSpec((1, C), lambda i: (0, 0)),
            pl.BlockSpec((1, C), lambda i: (0, 0)),
        ],
        out_specs=pl.BlockSpec((256, C), lambda i: (i, 0)),
        out_shape=jax.ShapeDtypeStruct((B * N, C), jnp.float32),
    )(x2d, cp, wc0t, w2t, bc2, b22)


def _c2_body(col_ref, val_ref, imp_ref):
    i = pl.program_id(1)
    row = lax.broadcasted_iota(jnp.int32, (256, 128), 0) + i * 256
    cols = col_ref[0]
    vals = val_ref[0]
    for s in range(N // 128):
        col = lax.broadcasted_iota(jnp.int32, (256, 128), 1) + s * 128
        imp = jnp.where(col == row, 1.0, 0.0).astype(jnp.float32)
        for k in range(K):
            imp = jnp.where(col == cols[:, k:k + 1], vals[:, k:k + 1], imp)
        imp_ref[0, :, pl.ds(s * 128, 128)] = imp


def _stage_c2(scol3, sval3):
    return pl.pallas_call(
        _c2_body,
        grid=(B, N // 256),
        in_specs=[
            pl.BlockSpec((1, 256, K), lambda b, i: (b, i, 0)),
            pl.BlockSpec((1, 256, K), lambda b, i: (b, i, 0)),
        ],
        out_specs=pl.BlockSpec((1, 256, N), lambda b, i: (b, i, 0)),
        out_shape=jax.ShapeDtypeStruct((B, N, N), jnp.float32),
    )(scol3, sval3)


def kernel(x, adj_mat, W_conv, b_conv, W2, b2):
    adj = adj_mat.astype(jnp.int32)
    wg3 = jnp.transpose(W_conv[:, :, 1:], (1, 2, 0))
    wcat = jnp.concatenate([wg3[:, :, _half_channels(0)],
                            wg3[:, :, _half_channels(16)]],
                           axis=-1).reshape(C, K * C)
    wc0t = W_conv[:, :, 0].T
    w2t = W2.T
    bc2 = b_conv.reshape(1, C)
    b22 = b2.reshape(1, C)

    x2d = x.reshape(B * N, C)
    yg_flat = _stage_a(x2d, wcat)

    gidx = (adj + (jnp.arange(B, dtype=jnp.int32) * N)[:, None, None]
            + (jnp.arange(K, dtype=jnp.int32) * (B * N))[None, None, :])
    cp = _stage_b(yg_flat, gidx.reshape(-1))

    cnn2d = _stage_c1(x2d, cp, wc0t, w2t, bc2, b22)

    rowid = jnp.arange(B * N, dtype=jnp.int32)[:, None]
    keys = (rowid * N + adj.reshape(B * N, K)).reshape(-1)
    perm = jnp.arange(B * N * K, dtype=jnp.int32)
    sk, sp = lax.sort_key_val(keys, perm, is_stable=False)
    nw = cnn2d[:, :K].reshape(-1)
    sv = nw[sp]
    scol = sk.reshape(B * N, K) - rowid * N
    imp = _stage_c2(scol.reshape(B, N, K), sv.reshape(B, N, K))

    return (cnn2d.reshape(B, N, C), imp)
